# baseline (speedup 1.0000x reference)
import functools

import jax
import jax.numpy as jnp
from jax import lax
from jax.experimental import pallas as pl
from jax.experimental.pallas import tpu as pltpu
from jax.experimental.pallas import tpu_sc as plsc

N = 10000
E = 320000
D = 128
NUM_CLASSES = 16

NUM_TILES = 16
CHUNK = 128
IDX_GRP = 8
CHUNKS_PER_TILE = 160
IDX_GRPS = CHUNKS_PER_TILE // IDX_GRP
TILE_EDGES = CHUNKS_PER_TILE * CHUNK
E_PAD = NUM_TILES * TILE_EDGES
N_PAD = 10112
ROWS_PER_TILE = N_PAD // NUM_TILES
_ROW_SPANS = [(q, min(CHUNK, ROWS_PER_TILE - q))
              for q in range(0, ROWS_PER_TILE, CHUNK)]
ROW_BLOCK = 1000
NUM_ROW_BLOCKS = N // ROW_BLOCK


def _sc_mesh():
    return plsc.VectorSubcoreMesh(core_axis_name="c", subcore_axis_name="s")


def _degrees(sidx, z128, ones128):
    @functools.partial(
        pl.kernel,
        out_type=jax.ShapeDtypeStruct((2, N_PAD, D), jnp.float32),
        mesh=_sc_mesh(),
        scratch_types=[
            pltpu.VMEM((IDX_GRP, CHUNK), jnp.int32),
            pltpu.VMEM((CHUNK, D), jnp.float32),
            pltpu.VMEM((CHUNK, D), jnp.float32),
            pltpu.VMEM_SHARED((N_PAD, D), jnp.float32),
            pltpu.SemaphoreType.DMA,
        ],
    )
    def body(sidx_h, z128_h, ones_h, deg_h, idxs_v, rows_v, ones_v, deg_sh,
             ssem):
        c = lax.axis_index("c")
        s = lax.axis_index("s")
        row0 = s * ROWS_PER_TILE
        pltpu.sync_copy(z128_h, rows_v)
        pltpu.sync_copy(ones_h, ones_v)
        for q0, qn in _ROW_SPANS:
            pltpu.sync_copy(rows_v.at[pl.ds(0, qn)],
                            deg_sh.at[pl.ds(row0 + q0, qn)])
        plsc.subcore_barrier()

        @pl.loop(0, IDX_GRPS)
        def _(g):
            pltpu.sync_copy(sidx_h.at[c, s, pl.ds(g * IDX_GRP, IDX_GRP)], idxs_v)
            descs = []
            for k in range(IDX_GRP):
                descs.append(pltpu.async_copy(
                    ones_v, deg_sh.at[idxs_v.at[k]], ssem, add=True))
            for d in descs:
                d.wait()

        plsc.subcore_barrier()
        for q0, qn in _ROW_SPANS:
            pltpu.sync_copy(deg_sh.at[pl.ds(row0 + q0, qn)], rows_v.at[pl.ds(0, qn)])
            pltpu.sync_copy(rows_v.at[pl.ds(0, qn)],
                            deg_h.at[c, pl.ds(row0 + q0, qn)])

    return body(sidx, z128, ones128)


def _propagate(table2, gidx, sidx, z128):
    @functools.partial(
        pl.kernel,
        out_type=jax.ShapeDtypeStruct((2, N_PAD, D), jnp.float32),
        mesh=_sc_mesh(),
        scratch_types=[
            pltpu.VMEM((IDX_GRP, CHUNK), jnp.int32),
            pltpu.VMEM((IDX_GRP, CHUNK), jnp.int32),
            pltpu.VMEM((2, CHUNK, D), jnp.float32),
            pltpu.VMEM_SHARED((N_PAD, D), jnp.float32),
            pltpu.SemaphoreType.DMA,
            pltpu.SemaphoreType.DMA,
            pltpu.SemaphoreType.DMA,
            pltpu.SemaphoreType.DMA,
            pltpu.SemaphoreType.DMA,
            pltpu.SemaphoreType.DMA,
        ],
    )
    def body(table_h, gidx_h, sidx_h, z128_h, agg_h,
             idxg_v, idxs_v, rows_v, acc_sh,
             gsem0a, gsem0b, gsem1a, gsem1b, ssem0, ssem1):
        c = lax.axis_index("c")
        s = lax.axis_index("s")
        row0 = s * ROWS_PER_TILE
        gsems = ((gsem0a, gsem0b), (gsem1a, gsem1b))
        ssems = (ssem0, ssem1)
        H = CHUNK // 2
        pltpu.sync_copy(z128_h, rows_v.at[0])
        for q0, qn in _ROW_SPANS:
            pltpu.sync_copy(rows_v.at[0, pl.ds(0, qn)],
                            acc_sh.at[pl.ds(row0 + q0, qn)])
        plsc.subcore_barrier()

        @pl.loop(0, IDX_GRPS)
        def _(g):
            pltpu.sync_copy(gidx_h.at[c, s, pl.ds(g * IDX_GRP, IDX_GRP)], idxg_v)
            pltpu.sync_copy(sidx_h.at[c, s, pl.ds(g * IDX_GRP, IDX_GRP)], idxs_v)
            def fire_gather(k, b):
                return (
                    pltpu.async_copy(table_h.at[idxg_v.at[k, pl.ds(0, H)]],
                                     rows_v.at[b, pl.ds(0, H)], gsems[b][0]),
                    pltpu.async_copy(table_h.at[idxg_v.at[k, pl.ds(H, H)]],
                                     rows_v.at[b, pl.ds(H, H)], gsems[b][1]),
                )

            gath = [None] * IDX_GRP
            scat = [None, None]
            gath[0] = fire_gather(0, 0)
            for k in range(IDX_GRP):
                b = k % 2
                for d in gath[k]:
                    d.wait()
                if k + 1 < IDX_GRP:
                    if scat[1 - b] is not None:
                        scat[1 - b].wait()
                    gath[k + 1] = fire_gather(k + 1, 1 - b)
                scat[b] = pltpu.async_copy(
                    rows_v.at[b], acc_sh.at[idxs_v.at[k]], ssems[b], add=True)
            scat[0].wait()
            scat[1].wait()

        plsc.subcore_barrier()
        for q0, qn in _ROW_SPANS:
            pltpu.sync_copy(acc_sh.at[pl.ds(row0 + q0, qn)],
                            rows_v.at[0, pl.ds(0, qn)])
            pltpu.sync_copy(rows_v.at[0, pl.ds(0, qn)],
                            agg_h.at[c, pl.ds(row0 + q0, qn)])

    return body(table2, gidx, sidx, z128)


def _dense_layer(x, w, b2d):
    def body(x_ref, w_ref, b_ref, o_ref):
        acc = jnp.dot(x_ref[...], w_ref[...],
                      preferred_element_type=jnp.float32)
        o_ref[...] = jnp.maximum(acc + b_ref[...], 0.0)

    return pl.pallas_call(
        body,
        grid=(NUM_ROW_BLOCKS,),
        in_specs=[
            pl.BlockSpec((ROW_BLOCK, D), lambda i: (i, 0)),
            pl.BlockSpec((D, D), lambda i: (0, 0)),
            pl.BlockSpec((1, D), lambda i: (0, 0)),
        ],
        out_specs=pl.BlockSpec((ROW_BLOCK, D), lambda i: (i, 0)),
        out_shape=jax.ShapeDtypeStruct((N, D), jnp.float32),
    )(x, w, b2d)


def _mlp_pair(h_prev, agg, deg, w0s, w1s):
    hd = h_prev.shape[0]

    def body(h_ref, a_ref, g_ref, w0_ref, w1_ref, o_ref):
        h = h_ref[0]
        a = a_ref[0]
        inv = 1.0 / jnp.maximum(g_ref[0][:, 0:1], 1.0)
        an = a * inv
        t = jnp.dot(h, w0_ref[0, :D, :], preferred_element_type=jnp.float32)
        t += jnp.dot(an, w0_ref[0, D:, :], preferred_element_type=jnp.float32)
        t = jnp.maximum(t, 0.0)
        o_ref[0] = jnp.maximum(
            jnp.dot(t, w1_ref[0], preferred_element_type=jnp.float32), 0.0)

    return pl.pallas_call(
        body,
        grid=(2, NUM_ROW_BLOCKS),
        in_specs=[
            pl.BlockSpec((1, ROW_BLOCK, D),
                         lambda d, i: (d if hd == 2 else 0, i, 0)),
            pl.BlockSpec((1, ROW_BLOCK, D), lambda d, i: (d, i, 0)),
            pl.BlockSpec((1, ROW_BLOCK, D), lambda d, i: (d, i, 0)),
            pl.BlockSpec((1, 2 * D, D), lambda d, i: (d, 0, 0)),
            pl.BlockSpec((1, D, D), lambda d, i: (d, 0, 0)),
        ],
        out_specs=pl.BlockSpec((1, ROW_BLOCK, D), lambda d, i: (d, i, 0)),
        out_shape=jax.ShapeDtypeStruct((2, N, D), jnp.float32),
    )(h_prev, agg, deg, w0s, w1s)


def _final_layer(dense, h1, h2, maskf, fc2_w, fc2_b2d):
    def body(d_ref, h1_ref, h2_ref, m_ref, w_ref, b_ref, o_ref):
        rep = jnp.dot(d_ref[...], w_ref[0:D, :],
                      preferred_element_type=jnp.float32)
        rep += jnp.dot(h1_ref[0], w_ref[D:2 * D, :],
                       preferred_element_type=jnp.float32)
        rep += jnp.dot(h2_ref[0], w_ref[2 * D:3 * D, :],
                       preferred_element_type=jnp.float32)
        rep += jnp.dot(h1_ref[1], w_ref[3 * D:4 * D, :],
                       preferred_element_type=jnp.float32)
        rep += jnp.dot(h2_ref[1], w_ref[4 * D:5 * D, :],
                       preferred_element_type=jnp.float32)
        rep = jnp.where(m_ref[...] > 0.0, rep, 0.0) + b_ref[...]
        mx = jnp.max(rep, axis=-1, keepdims=True)
        lse = jnp.log(jnp.sum(jnp.exp(rep - mx), axis=-1, keepdims=True)) + mx
        o_ref[...] = rep - lse

    return pl.pallas_call(
        body,
        grid=(NUM_ROW_BLOCKS,),
        in_specs=[
            pl.BlockSpec((ROW_BLOCK, D), lambda i: (i, 0)),
            pl.BlockSpec((2, ROW_BLOCK, D), lambda i: (0, i, 0)),
            pl.BlockSpec((2, ROW_BLOCK, D), lambda i: (0, i, 0)),
            pl.BlockSpec((ROW_BLOCK, 1), lambda i: (i, 0)),
            pl.BlockSpec((5 * D, NUM_CLASSES), lambda i: (0, 0)),
            pl.BlockSpec((1, NUM_CLASSES), lambda i: (0, 0)),
        ],
        out_specs=pl.BlockSpec((ROW_BLOCK, NUM_CLASSES), lambda i: (i, 0)),
        out_shape=jax.ShapeDtypeStruct((N, NUM_CLASSES), jnp.float32),
    )(dense, h1, h2, maskf, fc2_w, fc2_b2d)


def kernel(x, edge_index, mask, fc1_w, fc1_b, mw_0_0, mw_0_1, mw_1_0, mw_1_1,
           rw_0_0, rw_0_1, rw_1_0, rw_1_1, fc2_w, fc2_b):
    src = edge_index[0]
    dst = edge_index[1]
    pad = E_PAD - E
    zpad = jnp.zeros((pad,), jnp.int32)
    npad = jnp.full((pad,), N, jnp.int32)
    src_p = jnp.concatenate([src, zpad])
    dst_p = jnp.concatenate([dst, zpad])
    idx_shape = (2, NUM_TILES, CHUNKS_PER_TILE, CHUNK)
    gidx1 = jnp.stack([src_p, dst_p]).reshape(idx_shape)
    gidx2 = jnp.stack([src_p, dst_p + N]).reshape(idx_shape)
    sidx = jnp.stack(
        [jnp.concatenate([dst, npad]), jnp.concatenate([src, npad])]
    ).reshape(idx_shape)
    z128 = jnp.zeros((CHUNK, D), jnp.float32)
    ones128 = jnp.ones((CHUNK, D), jnp.float32)
    maskf = mask.astype(jnp.float32).reshape(N, 1)

    dense = _dense_layer(x, fc1_w, fc1_b.reshape(1, D))
    deg = _degrees(sidx, z128, ones128)
    agg1 = _propagate(dense, gidx1, sidx, z128)
    h1 = _mlp_pair(dense.reshape(1, N, D), agg1, deg,
                   jnp.stack([mw_0_0, rw_0_0]), jnp.stack([mw_0_1, rw_0_1]))
    agg2 = _propagate(h1.reshape(2 * N, D), gidx2, sidx, z128)
    h2 = _mlp_pair(h1, agg2, deg,
                   jnp.stack([mw_1_0, rw_1_0]), jnp.stack([mw_1_1, rw_1_1]))
    return _final_layer(dense, h1, h2, maskf, fc2_w, fc2_b.reshape(1, NUM_CLASSES))


# speedup vs baseline: 1.0230x; 1.0230x over previous
import functools

import jax
import jax.numpy as jnp
from jax import lax
from jax.experimental import pallas as pl
from jax.experimental.pallas import tpu as pltpu
from jax.experimental.pallas import tpu_sc as plsc

N = 10000
E = 320000
D = 128
NUM_CLASSES = 16

NUM_TILES = 16
CHUNK = 128
IDX_GRP = 8
CHUNKS_PER_TILE = 160
IDX_GRPS = CHUNKS_PER_TILE // IDX_GRP
TILE_EDGES = CHUNKS_PER_TILE * CHUNK
E_PAD = NUM_TILES * TILE_EDGES
N_PAD = 10112
ROWS_PER_TILE = N_PAD // NUM_TILES
_ROW_SPANS = [(q, min(CHUNK, ROWS_PER_TILE - q))
              for q in range(0, ROWS_PER_TILE, CHUNK)]
ROW_BLOCK = 1000
NUM_ROW_BLOCKS = N // ROW_BLOCK


def _sc_mesh():
    return plsc.VectorSubcoreMesh(core_axis_name="c", subcore_axis_name="s")


def _degrees(sidx, z128, ones128):
    @functools.partial(
        pl.kernel,
        out_type=jax.ShapeDtypeStruct((2, N_PAD, D), jnp.float32),
        mesh=_sc_mesh(),
        scratch_types=[
            pltpu.VMEM((IDX_GRP, CHUNK), jnp.int32),
            pltpu.VMEM((CHUNK, D), jnp.float32),
            pltpu.VMEM((CHUNK, D), jnp.float32),
            pltpu.VMEM_SHARED((N_PAD, D), jnp.float32),
            pltpu.SemaphoreType.DMA,
        ],
    )
    def body(sidx_h, z128_h, ones_h, deg_h, idxs_v, rows_v, ones_v, deg_sh,
             ssem):
        c = lax.axis_index("c")
        s = lax.axis_index("s")
        row0 = s * ROWS_PER_TILE
        pltpu.sync_copy(z128_h, rows_v)
        pltpu.sync_copy(ones_h, ones_v)
        for q0, qn in _ROW_SPANS:
            pltpu.sync_copy(rows_v.at[pl.ds(0, qn)],
                            deg_sh.at[pl.ds(row0 + q0, qn)])
        plsc.subcore_barrier()

        @pl.loop(0, IDX_GRPS)
        def _(g):
            pltpu.sync_copy(sidx_h.at[c, s, pl.ds(g * IDX_GRP, IDX_GRP)], idxs_v)
            descs = []
            for k in range(IDX_GRP):
                descs.append(pltpu.async_copy(
                    ones_v, deg_sh.at[idxs_v.at[k]], ssem, add=True))
            for d in descs:
                d.wait()

        plsc.subcore_barrier()
        for q0, qn in _ROW_SPANS:
            pltpu.sync_copy(deg_sh.at[pl.ds(row0 + q0, qn)], rows_v.at[pl.ds(0, qn)])
            pltpu.sync_copy(rows_v.at[pl.ds(0, qn)],
                            deg_h.at[c, pl.ds(row0 + q0, qn)])

    return body(sidx, z128, ones128)


def _propagate(table2, gidx, sidx, z128):
    @functools.partial(
        pl.kernel,
        out_type=jax.ShapeDtypeStruct((2, N_PAD, D), jnp.float32),
        mesh=_sc_mesh(),
        scratch_types=[
            pltpu.VMEM((IDX_GRP, CHUNK), jnp.int32),
            pltpu.VMEM((IDX_GRP, CHUNK), jnp.int32),
            pltpu.VMEM((2, CHUNK, D), jnp.float32),
            pltpu.VMEM_SHARED((N_PAD, D), jnp.float32),
            pltpu.SemaphoreType.DMA,
            pltpu.SemaphoreType.DMA,
            pltpu.SemaphoreType.DMA,
            pltpu.SemaphoreType.DMA,
            pltpu.SemaphoreType.DMA,
            pltpu.SemaphoreType.DMA,
        ],
    )
    def body(table_h, gidx_h, sidx_h, z128_h, agg_h,
             idxg_v, idxs_v, rows_v, acc_sh,
             gsem0a, gsem0b, gsem1a, gsem1b, ssem0, ssem1):
        c = lax.axis_index("c")
        s = lax.axis_index("s")
        row0 = s * ROWS_PER_TILE
        gsems = ((gsem0a, gsem0b), (gsem1a, gsem1b))
        ssems = (ssem0, ssem1)
        H = CHUNK // 2
        pltpu.sync_copy(z128_h, rows_v.at[0])
        for q0, qn in _ROW_SPANS:
            pltpu.sync_copy(rows_v.at[0, pl.ds(0, qn)],
                            acc_sh.at[pl.ds(row0 + q0, qn)])
        plsc.subcore_barrier()

        @pl.loop(0, IDX_GRPS)
        def _(g):
            pltpu.sync_copy(gidx_h.at[c, s, pl.ds(g * IDX_GRP, IDX_GRP)], idxg_v)
            pltpu.sync_copy(sidx_h.at[c, s, pl.ds(g * IDX_GRP, IDX_GRP)], idxs_v)
            def fire_gather(k, b):
                return (
                    pltpu.async_copy(table_h.at[idxg_v.at[k, pl.ds(0, H)]],
                                     rows_v.at[b, pl.ds(0, H)], gsems[b][0]),
                    pltpu.async_copy(table_h.at[idxg_v.at[k, pl.ds(H, H)]],
                                     rows_v.at[b, pl.ds(H, H)], gsems[b][1]),
                )

            gath = [None] * IDX_GRP
            scat = [None, None]
            gath[0] = fire_gather(0, 0)
            gath[1] = fire_gather(1, 1)
            for k in range(IDX_GRP):
                b = k % 2
                for d in gath[k]:
                    d.wait()
                scat[b] = pltpu.async_copy(
                    rows_v.at[b], acc_sh.at[idxs_v.at[k]], ssems[b], add=True)
                if k + 2 < IDX_GRP:
                    scat[b].wait()
                    scat[b] = None
                    gath[k + 2] = fire_gather(k + 2, b)
            for d in scat:
                if d is not None:
                    d.wait()

        plsc.subcore_barrier()
        for q0, qn in _ROW_SPANS:
            pltpu.sync_copy(acc_sh.at[pl.ds(row0 + q0, qn)],
                            rows_v.at[0, pl.ds(0, qn)])
            pltpu.sync_copy(rows_v.at[0, pl.ds(0, qn)],
                            agg_h.at[c, pl.ds(row0 + q0, qn)])

    return body(table2, gidx, sidx, z128)


def _dense_layer(x, w, b2d):
    def body(x_ref, w_ref, b_ref, o_ref):
        acc = jnp.dot(x_ref[...], w_ref[...],
                      preferred_element_type=jnp.float32)
        o_ref[...] = jnp.maximum(acc + b_ref[...], 0.0)

    return pl.pallas_call(
        body,
        grid=(NUM_ROW_BLOCKS,),
        in_specs=[
            pl.BlockSpec((ROW_BLOCK, D), lambda i: (i, 0)),
            pl.BlockSpec((D, D), lambda i: (0, 0)),
            pl.BlockSpec((1, D), lambda i: (0, 0)),
        ],
        out_specs=pl.BlockSpec((ROW_BLOCK, D), lambda i: (i, 0)),
        out_shape=jax.ShapeDtypeStruct((N, D), jnp.float32),
    )(x, w, b2d)


def _mlp_pair(h_prev, agg, deg, w0s, w1s):
    hd = h_prev.shape[0]

    def body(h_ref, a_ref, g_ref, w0_ref, w1_ref, o_ref):
        h = h_ref[0]
        a = a_ref[0]
        inv = 1.0 / jnp.maximum(g_ref[0][:, 0:1], 1.0)
        an = a * inv
        t = jnp.dot(h, w0_ref[0, :D, :], preferred_element_type=jnp.float32)
        t += jnp.dot(an, w0_ref[0, D:, :], preferred_element_type=jnp.float32)
        t = jnp.maximum(t, 0.0)
        o_ref[0] = jnp.maximum(
            jnp.dot(t, w1_ref[0], preferred_element_type=jnp.float32), 0.0)

    return pl.pallas_call(
        body,
        grid=(2, NUM_ROW_BLOCKS),
        in_specs=[
            pl.BlockSpec((1, ROW_BLOCK, D),
                         lambda d, i: (d if hd == 2 else 0, i, 0)),
            pl.BlockSpec((1, ROW_BLOCK, D), lambda d, i: (d, i, 0)),
            pl.BlockSpec((1, ROW_BLOCK, D), lambda d, i: (d, i, 0)),
            pl.BlockSpec((1, 2 * D, D), lambda d, i: (d, 0, 0)),
            pl.BlockSpec((1, D, D), lambda d, i: (d, 0, 0)),
        ],
        out_specs=pl.BlockSpec((1, ROW_BLOCK, D), lambda d, i: (d, i, 0)),
        out_shape=jax.ShapeDtypeStruct((2, N, D), jnp.float32),
    )(h_prev, agg, deg, w0s, w1s)


def _final_layer(dense, h1, h2, maskf, fc2_w, fc2_b2d):
    def body(d_ref, h1_ref, h2_ref, m_ref, w_ref, b_ref, o_ref):
        rep = jnp.dot(d_ref[...], w_ref[0:D, :],
                      preferred_element_type=jnp.float32)
        rep += jnp.dot(h1_ref[0], w_ref[D:2 * D, :],
                       preferred_element_type=jnp.float32)
        rep += jnp.dot(h2_ref[0], w_ref[2 * D:3 * D, :],
                       preferred_element_type=jnp.float32)
        rep += jnp.dot(h1_ref[1], w_ref[3 * D:4 * D, :],
                       preferred_element_type=jnp.float32)
        rep += jnp.dot(h2_ref[1], w_ref[4 * D:5 * D, :],
                       preferred_element_type=jnp.float32)
        rep = jnp.where(m_ref[...] > 0.0, rep, 0.0) + b_ref[...]
        mx = jnp.max(rep, axis=-1, keepdims=True)
        lse = jnp.log(jnp.sum(jnp.exp(rep - mx), axis=-1, keepdims=True)) + mx
        o_ref[...] = rep - lse

    return pl.pallas_call(
        body,
        grid=(NUM_ROW_BLOCKS,),
        in_specs=[
            pl.BlockSpec((ROW_BLOCK, D), lambda i: (i, 0)),
            pl.BlockSpec((2, ROW_BLOCK, D), lambda i: (0, i, 0)),
            pl.BlockSpec((2, ROW_BLOCK, D), lambda i: (0, i, 0)),
            pl.BlockSpec((ROW_BLOCK, 1), lambda i: (i, 0)),
            pl.BlockSpec((5 * D, NUM_CLASSES), lambda i: (0, 0)),
            pl.BlockSpec((1, NUM_CLASSES), lambda i: (0, 0)),
        ],
        out_specs=pl.BlockSpec((ROW_BLOCK, NUM_CLASSES), lambda i: (i, 0)),
        out_shape=jax.ShapeDtypeStruct((N, NUM_CLASSES), jnp.float32),
    )(dense, h1, h2, maskf, fc2_w, fc2_b2d)


def kernel(x, edge_index, mask, fc1_w, fc1_b, mw_0_0, mw_0_1, mw_1_0, mw_1_1,
           rw_0_0, rw_0_1, rw_1_0, rw_1_1, fc2_w, fc2_b):
    src = edge_index[0]
    dst = edge_index[1]
    pad = E_PAD - E
    zpad = jnp.zeros((pad,), jnp.int32)
    npad = jnp.full((pad,), N, jnp.int32)
    src_p = jnp.concatenate([src, zpad])
    dst_p = jnp.concatenate([dst, zpad])
    idx_shape = (2, NUM_TILES, CHUNKS_PER_TILE, CHUNK)
    gidx1 = jnp.stack([src_p, dst_p]).reshape(idx_shape)
    gidx2 = jnp.stack([src_p, dst_p + N]).reshape(idx_shape)
    sidx = jnp.stack(
        [jnp.concatenate([dst, npad]), jnp.concatenate([src, npad])]
    ).reshape(idx_shape)
    z128 = jnp.zeros((CHUNK, D), jnp.float32)
    ones128 = jnp.ones((CHUNK, D), jnp.float32)
    maskf = mask.astype(jnp.float32).reshape(N, 1)

    dense = _dense_layer(x, fc1_w, fc1_b.reshape(1, D))
    deg = _degrees(sidx, z128, ones128)
    agg1 = _propagate(dense, gidx1, sidx, z128)
    h1 = _mlp_pair(dense.reshape(1, N, D), agg1, deg,
                   jnp.stack([mw_0_0, rw_0_0]), jnp.stack([mw_0_1, rw_0_1]))
    agg2 = _propagate(h1.reshape(2 * N, D), gidx2, sidx, z128)
    h2 = _mlp_pair(h1, agg2, deg,
                   jnp.stack([mw_1_0, rw_1_0]), jnp.stack([mw_1_1, rw_1_1]))
    return _final_layer(dense, h1, h2, maskf, fc2_w, fc2_b.reshape(1, NUM_CLASSES))


# speedup vs baseline: 1.1947x; 1.1678x over previous
import functools

import jax
import jax.numpy as jnp
from jax import lax
from jax.experimental import pallas as pl
from jax.experimental.pallas import tpu as pltpu
from jax.experimental.pallas import tpu_sc as plsc

N = 10000
E = 320000
D = 128
NUM_CLASSES = 16

NUM_TILES = 16
CHUNK = 128
IDX_GRP = 8
CHUNKS_PER_TILE = 160
IDX_GRPS = CHUNKS_PER_TILE // IDX_GRP
TILE_EDGES = CHUNKS_PER_TILE * CHUNK
E_PAD = NUM_TILES * TILE_EDGES
N_PAD = 10112
ROWS_PER_TILE = N_PAD // NUM_TILES
_ROW_SPANS = [(q, min(CHUNK, ROWS_PER_TILE - q))
              for q in range(0, ROWS_PER_TILE, CHUNK)]
ROW_BLOCK = 1000
NUM_ROW_BLOCKS = N // ROW_BLOCK


def _sc_mesh():
    return plsc.VectorSubcoreMesh(core_axis_name="c", subcore_axis_name="s")


def _degrees(sidx, z128, ones128):
    @functools.partial(
        pl.kernel,
        out_type=jax.ShapeDtypeStruct((2, N_PAD, D), jnp.float32),
        mesh=_sc_mesh(),
        scratch_types=[
            pltpu.VMEM((IDX_GRP, CHUNK), jnp.int32),
            pltpu.VMEM((CHUNK, D), jnp.float32),
            pltpu.VMEM((CHUNK, D), jnp.float32),
            pltpu.VMEM_SHARED((N_PAD, D), jnp.float32),
            pltpu.SemaphoreType.DMA,
        ],
    )
    def body(sidx_h, z128_h, ones_h, deg_h, idxs_v, rows_v, ones_v, deg_sh,
             ssem):
        c = lax.axis_index("c")
        s = lax.axis_index("s")
        row0 = s * ROWS_PER_TILE
        pltpu.sync_copy(z128_h, rows_v)
        pltpu.sync_copy(ones_h, ones_v)
        for q0, qn in _ROW_SPANS:
            pltpu.sync_copy(rows_v.at[pl.ds(0, qn)],
                            deg_sh.at[pl.ds(row0 + q0, qn)])
        plsc.subcore_barrier()

        @pl.loop(0, IDX_GRPS)
        def _(g):
            pltpu.sync_copy(sidx_h.at[c, s, pl.ds(g * IDX_GRP, IDX_GRP)], idxs_v)
            descs = []
            for k in range(IDX_GRP):
                descs.append(pltpu.async_copy(
                    ones_v, deg_sh.at[idxs_v.at[k]], ssem, add=True))
            for d in descs:
                d.wait()

        plsc.subcore_barrier()
        for q0, qn in _ROW_SPANS:
            pltpu.sync_copy(deg_sh.at[pl.ds(row0 + q0, qn)], rows_v.at[pl.ds(0, qn)])
            pltpu.sync_copy(rows_v.at[pl.ds(0, qn)],
                            deg_h.at[c, pl.ds(row0 + q0, qn)])

    return body(sidx, z128, ones128)


def _propagate(table2, gidx, sidx, z128):
    @functools.partial(
        pl.kernel,
        out_type=jax.ShapeDtypeStruct((2, N_PAD, D), jnp.float32),
        mesh=_sc_mesh(),
        scratch_types=[
            pltpu.VMEM((IDX_GRP, CHUNK), jnp.int32),
            pltpu.VMEM((IDX_GRP, CHUNK), jnp.int32),
            pltpu.VMEM((2, CHUNK, D), jnp.float32),
            pltpu.VMEM_SHARED((N_PAD, D), jnp.float32),
            pltpu.SemaphoreType.DMA,
            pltpu.SemaphoreType.DMA,
            pltpu.SemaphoreType.DMA,
            pltpu.SemaphoreType.DMA,
            pltpu.SemaphoreType.DMA,
            pltpu.SemaphoreType.DMA,
        ],
    )
    def body(table_h, gidx_h, sidx_h, z128_h, agg_h,
             idxg_v, idxs_v, rows_v, acc_sh,
             gsem0a, gsem0b, gsem1a, gsem1b, ssem0, ssem1):
        c = lax.axis_index("c")
        s = lax.axis_index("s")
        row0 = s * ROWS_PER_TILE
        gsems = ((gsem0a, gsem0b), (gsem1a, gsem1b))
        ssems = (ssem0, ssem1)
        H = CHUNK // 2
        pltpu.sync_copy(z128_h, rows_v.at[0])
        for q0, qn in _ROW_SPANS:
            pltpu.sync_copy(rows_v.at[0, pl.ds(0, qn)],
                            acc_sh.at[pl.ds(row0 + q0, qn)])
        plsc.subcore_barrier()

        @pl.loop(0, IDX_GRPS)
        def _(g):
            pltpu.sync_copy(gidx_h.at[c, s, pl.ds(g * IDX_GRP, IDX_GRP)], idxg_v)
            pltpu.sync_copy(sidx_h.at[c, s, pl.ds(g * IDX_GRP, IDX_GRP)], idxs_v)
            def fire_gather(k, b):
                return (
                    pltpu.async_copy(table_h.at[idxg_v.at[k, pl.ds(0, H)]],
                                     rows_v.at[b, pl.ds(0, H)], gsems[b][0]),
                    pltpu.async_copy(table_h.at[idxg_v.at[k, pl.ds(H, H)]],
                                     rows_v.at[b, pl.ds(H, H)], gsems[b][1]),
                )

            gath = [None] * IDX_GRP
            scat = [None, None]
            gath[0] = fire_gather(0, 0)
            gath[1] = fire_gather(1, 1)
            for k in range(IDX_GRP):
                b = k % 2
                for d in gath[k]:
                    d.wait()
                scat[b] = pltpu.async_copy(
                    rows_v.at[b], acc_sh.at[idxs_v.at[k]], ssems[b], add=True)
                if k + 2 < IDX_GRP:
                    scat[b].wait()
                    scat[b] = None
                    gath[k + 2] = fire_gather(k + 2, b)
            for d in scat:
                if d is not None:
                    d.wait()

        plsc.subcore_barrier()
        for q0, qn in _ROW_SPANS:
            pltpu.sync_copy(acc_sh.at[pl.ds(row0 + q0, qn)],
                            rows_v.at[0, pl.ds(0, qn)])
            pltpu.sync_copy(rows_v.at[0, pl.ds(0, qn)],
                            agg_h.at[c, pl.ds(row0 + q0, qn)])

    return body(table2, gidx, sidx, z128)


def _dense_layer(x, w, b2d):
    def body(x_ref, w_ref, b_ref, o_ref):
        acc = jnp.dot(x_ref[...], w_ref[...],
                      preferred_element_type=jnp.float32)
        v = jnp.maximum(acc + b_ref[...], 0.0)
        o_ref[0] = v
        o_ref[1] = v

    return pl.pallas_call(
        body,
        grid=(NUM_ROW_BLOCKS,),
        in_specs=[
            pl.BlockSpec((ROW_BLOCK, D), lambda i: (i, 0)),
            pl.BlockSpec((D, D), lambda i: (0, 0)),
            pl.BlockSpec((1, D), lambda i: (0, 0)),
        ],
        out_specs=pl.BlockSpec((2, ROW_BLOCK, D), lambda i: (0, i, 0)),
        out_shape=jax.ShapeDtypeStruct((2, N, D), jnp.float32),
    )(x, w, b2d)


def _mlp_pair(h_prev, agg, deg, w0s, w1s):
    hd = h_prev.shape[0]

    def body(h_ref, a_ref, g_ref, w0_ref, w1_ref, o_ref):
        h = h_ref[0]
        a = a_ref[0]
        inv = 1.0 / jnp.maximum(g_ref[0][:, 0:1], 1.0)
        an = a * inv
        t = jnp.dot(h, w0_ref[0, :D, :], preferred_element_type=jnp.float32)
        t += jnp.dot(an, w0_ref[0, D:, :], preferred_element_type=jnp.float32)
        t = jnp.maximum(t, 0.0)
        o_ref[0] = jnp.maximum(
            jnp.dot(t, w1_ref[0], preferred_element_type=jnp.float32), 0.0)

    return pl.pallas_call(
        body,
        grid=(2, NUM_ROW_BLOCKS),
        in_specs=[
            pl.BlockSpec((1, ROW_BLOCK, D),
                         lambda d, i: (d if hd == 2 else 0, i, 0)),
            pl.BlockSpec((1, ROW_BLOCK, D), lambda d, i: (d, i, 0)),
            pl.BlockSpec((1, ROW_BLOCK, D), lambda d, i: (d, i, 0)),
            pl.BlockSpec((1, 2 * D, D), lambda d, i: (d, 0, 0)),
            pl.BlockSpec((1, D, D), lambda d, i: (d, 0, 0)),
        ],
        out_specs=pl.BlockSpec((1, ROW_BLOCK, D), lambda d, i: (d, i, 0)),
        out_shape=jax.ShapeDtypeStruct((2, N, D), jnp.float32),
    )(h_prev, agg, deg, w0s, w1s)


def _final_layer(dense, h1, h2, maskf, fc2_w, fc2_b2d):
    def body(d_ref, h1_ref, h2_ref, m_ref, w_ref, b_ref, o_ref):
        rep = jnp.dot(d_ref[...], w_ref[0:D, :],
                      preferred_element_type=jnp.float32)
        rep += jnp.dot(h1_ref[0], w_ref[D:2 * D, :],
                       preferred_element_type=jnp.float32)
        rep += jnp.dot(h2_ref[0], w_ref[2 * D:3 * D, :],
                       preferred_element_type=jnp.float32)
        rep += jnp.dot(h1_ref[1], w_ref[3 * D:4 * D, :],
                       preferred_element_type=jnp.float32)
        rep += jnp.dot(h2_ref[1], w_ref[4 * D:5 * D, :],
                       preferred_element_type=jnp.float32)
        rep = jnp.where(m_ref[...] > 0.0, rep, 0.0) + b_ref[...]
        mx = jnp.max(rep, axis=-1, keepdims=True)
        lse = jnp.log(jnp.sum(jnp.exp(rep - mx), axis=-1, keepdims=True)) + mx
        o_ref[...] = rep - lse

    return pl.pallas_call(
        body,
        grid=(NUM_ROW_BLOCKS,),
        in_specs=[
            pl.BlockSpec((ROW_BLOCK, D), lambda i: (i, 0)),
            pl.BlockSpec((2, ROW_BLOCK, D), lambda i: (0, i, 0)),
            pl.BlockSpec((2, ROW_BLOCK, D), lambda i: (0, i, 0)),
            pl.BlockSpec((ROW_BLOCK, 1), lambda i: (i, 0)),
            pl.BlockSpec((5 * D, NUM_CLASSES), lambda i: (0, 0)),
            pl.BlockSpec((1, NUM_CLASSES), lambda i: (0, 0)),
        ],
        out_specs=pl.BlockSpec((ROW_BLOCK, NUM_CLASSES), lambda i: (i, 0)),
        out_shape=jax.ShapeDtypeStruct((N, NUM_CLASSES), jnp.float32),
    )(dense, h1, h2, maskf, fc2_w, fc2_b2d)


def kernel(x, edge_index, mask, fc1_w, fc1_b, mw_0_0, mw_0_1, mw_1_0, mw_1_1,
           rw_0_0, rw_0_1, rw_1_0, rw_1_1, fc2_w, fc2_b):
    src = edge_index[0]
    dst = edge_index[1]
    pad = E_PAD - E
    zpad = jnp.zeros((pad,), jnp.int32)
    npad = jnp.full((pad,), N, jnp.int32)
    src_p = jnp.concatenate([src, zpad])
    dst_p = jnp.concatenate([dst, zpad])
    idx_shape = (2, NUM_TILES, CHUNKS_PER_TILE, CHUNK)
    gidx2 = jnp.stack([src_p, dst_p + N]).reshape(idx_shape)
    sidx = jnp.stack(
        [jnp.concatenate([dst, npad]), jnp.concatenate([src, npad])]
    ).reshape(idx_shape)
    z128 = jnp.zeros((CHUNK, D), jnp.float32)
    ones128 = jnp.ones((CHUNK, D), jnp.float32)
    maskf = mask.astype(jnp.float32).reshape(N, 1)

    dense2 = _dense_layer(x, fc1_w, fc1_b.reshape(1, D))
    deg = _degrees(sidx, z128, ones128)
    agg1 = _propagate(dense2.reshape(2 * N, D), gidx2, sidx, z128)
    h1 = _mlp_pair(dense2, agg1, deg,
                   jnp.stack([mw_0_0, rw_0_0]), jnp.stack([mw_0_1, rw_0_1]))
    agg2 = _propagate(h1.reshape(2 * N, D), gidx2, sidx, z128)
    h2 = _mlp_pair(h1, agg2, deg,
                   jnp.stack([mw_1_0, rw_1_0]), jnp.stack([mw_1_1, rw_1_1]))
    return _final_layer(dense2[0], h1, h2, maskf, fc2_w,
                        fc2_b.reshape(1, NUM_CLASSES))


# speedup vs baseline: 1.7882x; 1.4968x over previous
import functools

import jax
import jax.numpy as jnp
from jax import lax
from jax.experimental import pallas as pl
from jax.experimental.pallas import tpu as pltpu
from jax.experimental.pallas import tpu_sc as plsc

N = 10000
E = 320000
D = 128
NUM_CLASSES = 16

NUM_TILES = 16
CHUNK = 120
IDX_GRP = 8
CHUNKS_PER_TILE = 168
IDX_GRPS = CHUNKS_PER_TILE // IDX_GRP
TILE_EDGES = CHUNKS_PER_TILE * CHUNK
E_PAD = NUM_TILES * TILE_EDGES
N_PAD = 10112
ROWS_PER_TILE = N_PAD // NUM_TILES
_ROW_SPANS = [(q, min(CHUNK, ROWS_PER_TILE - q))
              for q in range(0, ROWS_PER_TILE, CHUNK)]
ROW_BLOCK = 1000
NUM_ROW_BLOCKS = N // ROW_BLOCK


def _sc_mesh():
    return plsc.VectorSubcoreMesh(core_axis_name="c", subcore_axis_name="s")


def _degrees(sidx, z128, ones128):
    @functools.partial(
        pl.kernel,
        out_type=jax.ShapeDtypeStruct((2, N_PAD, D), jnp.float32),
        mesh=_sc_mesh(),
        scratch_types=[
            pltpu.VMEM((IDX_GRP, CHUNK), jnp.int32),
            pltpu.VMEM((CHUNK, D), jnp.float32),
            pltpu.VMEM((CHUNK, D), jnp.float32),
            pltpu.VMEM_SHARED((N_PAD, D), jnp.float32),
            pltpu.SemaphoreType.DMA,
        ],
    )
    def body(sidx_h, z128_h, ones_h, deg_h, idxs_v, rows_v, ones_v, deg_sh,
             ssem):
        c = lax.axis_index("c")
        s = lax.axis_index("s")
        row0 = s * ROWS_PER_TILE
        pltpu.sync_copy(z128_h, rows_v)
        pltpu.sync_copy(ones_h, ones_v)
        for q0, qn in _ROW_SPANS:
            pltpu.sync_copy(rows_v.at[pl.ds(0, qn)],
                            deg_sh.at[pl.ds(row0 + q0, qn)])
        plsc.subcore_barrier()

        @pl.loop(0, IDX_GRPS)
        def _(g):
            pltpu.sync_copy(sidx_h.at[c, s, pl.ds(g * IDX_GRP, IDX_GRP)], idxs_v)
            descs = []
            for k in range(IDX_GRP):
                descs.append(pltpu.async_copy(
                    ones_v, deg_sh.at[idxs_v.at[k]], ssem, add=True))
            for d in descs:
                d.wait()

        plsc.subcore_barrier()
        for q0, qn in _ROW_SPANS:
            pltpu.sync_copy(deg_sh.at[pl.ds(row0 + q0, qn)], rows_v.at[pl.ds(0, qn)])
            pltpu.sync_copy(rows_v.at[pl.ds(0, qn)],
                            deg_h.at[c, pl.ds(row0 + q0, qn)])

    return body(sidx, z128, ones128)


def _propagate(table2, gidx, sidx, z128):
    @functools.partial(
        pl.kernel,
        out_type=jax.ShapeDtypeStruct((2, N_PAD, D), jnp.float32),
        mesh=_sc_mesh(),
        scratch_types=[
            pltpu.VMEM((IDX_GRP, CHUNK), jnp.int32),
            pltpu.VMEM((IDX_GRP, CHUNK), jnp.int32),
            pltpu.VMEM((3, CHUNK, D), jnp.float32),
            pltpu.VMEM_SHARED((N_PAD, D), jnp.float32),
            pltpu.SemaphoreType.DMA,
            pltpu.SemaphoreType.DMA,
            pltpu.SemaphoreType.DMA,
            pltpu.SemaphoreType.DMA,
            pltpu.SemaphoreType.DMA,
            pltpu.SemaphoreType.DMA,
        ],
    )
    def body(table_h, gidx_h, sidx_h, z128_h, agg_h,
             idxg_v, idxs_v, rows_v, acc_sh,
             gsem0, gsem1, gsem2, ssem0, ssem1, ssem2):
        c = lax.axis_index("c")
        s = lax.axis_index("s")
        row0 = s * ROWS_PER_TILE
        gsems = (gsem0, gsem1, gsem2)
        ssems = (ssem0, ssem1, ssem2)
        pltpu.sync_copy(z128_h, rows_v.at[0])
        for q0, qn in _ROW_SPANS:
            pltpu.sync_copy(rows_v.at[0, pl.ds(0, qn)],
                            acc_sh.at[pl.ds(row0 + q0, qn)])
        plsc.subcore_barrier()

        @pl.loop(0, IDX_GRPS)
        def _(g):
            pltpu.sync_copy(gidx_h.at[c, s, pl.ds(g * IDX_GRP, IDX_GRP)], idxg_v)
            pltpu.sync_copy(sidx_h.at[c, s, pl.ds(g * IDX_GRP, IDX_GRP)], idxs_v)
            def fire_gather(k, b):
                return pltpu.async_copy(
                    table_h.at[idxg_v.at[k]], rows_v.at[b], gsems[b])

            gath = [None] * IDX_GRP
            scat = [None, None, None]
            gath[0] = fire_gather(0, 0)
            gath[1] = fire_gather(1, 1)
            for k in range(IDX_GRP):
                b = k % 3
                gath[k].wait()
                if k + 2 < IDX_GRP:
                    b2 = (k + 2) % 3
                    if scat[b2] is not None:
                        scat[b2].wait()
                        scat[b2] = None
                    gath[k + 2] = fire_gather(k + 2, b2)
                scat[b] = pltpu.async_copy(
                    rows_v.at[b], acc_sh.at[idxs_v.at[k]], ssems[b], add=True)
            for d in scat:
                if d is not None:
                    d.wait()

        plsc.subcore_barrier()
        for q0, qn in _ROW_SPANS:
            pltpu.sync_copy(acc_sh.at[pl.ds(row0 + q0, qn)],
                            rows_v.at[0, pl.ds(0, qn)])
            pltpu.sync_copy(rows_v.at[0, pl.ds(0, qn)],
                            agg_h.at[c, pl.ds(row0 + q0, qn)])

    return body(table2, gidx, sidx, z128)


def _dense_layer(x, w, b2d):
    def body(x_ref, w_ref, b_ref, o_ref):
        acc = jnp.dot(x_ref[...], w_ref[...],
                      preferred_element_type=jnp.float32)
        v = jnp.maximum(acc + b_ref[...], 0.0)
        o_ref[0] = v
        o_ref[1] = v

    return pl.pallas_call(
        body,
        grid=(NUM_ROW_BLOCKS,),
        in_specs=[
            pl.BlockSpec((ROW_BLOCK, D), lambda i: (i, 0)),
            pl.BlockSpec((D, D), lambda i: (0, 0)),
            pl.BlockSpec((1, D), lambda i: (0, 0)),
        ],
        out_specs=pl.BlockSpec((2, ROW_BLOCK, D), lambda i: (0, i, 0)),
        out_shape=jax.ShapeDtypeStruct((2, N, D), jnp.float32),
    )(x, w, b2d)


def _mlp_pair(h_prev, agg, deg, w0s, w1s):
    hd = h_prev.shape[0]

    def body(h_ref, a_ref, g_ref, w0_ref, w1_ref, o_ref):
        h = h_ref[0]
        a = a_ref[0]
        inv = 1.0 / jnp.maximum(g_ref[0][:, 0:1], 1.0)
        an = a * inv
        t = jnp.dot(h, w0_ref[0, :D, :], preferred_element_type=jnp.float32)
        t += jnp.dot(an, w0_ref[0, D:, :], preferred_element_type=jnp.float32)
        t = jnp.maximum(t, 0.0)
        o_ref[0] = jnp.maximum(
            jnp.dot(t, w1_ref[0], preferred_element_type=jnp.float32), 0.0)

    return pl.pallas_call(
        body,
        grid=(2, NUM_ROW_BLOCKS),
        in_specs=[
            pl.BlockSpec((1, ROW_BLOCK, D),
                         lambda d, i: (d if hd == 2 else 0, i, 0)),
            pl.BlockSpec((1, ROW_BLOCK, D), lambda d, i: (d, i, 0)),
            pl.BlockSpec((1, ROW_BLOCK, D), lambda d, i: (d, i, 0)),
            pl.BlockSpec((1, 2 * D, D), lambda d, i: (d, 0, 0)),
            pl.BlockSpec((1, D, D), lambda d, i: (d, 0, 0)),
        ],
        out_specs=pl.BlockSpec((1, ROW_BLOCK, D), lambda d, i: (d, i, 0)),
        out_shape=jax.ShapeDtypeStruct((2, N, D), jnp.float32),
    )(h_prev, agg, deg, w0s, w1s)


def _final_layer(dense, h1, h2, maskf, fc2_w, fc2_b2d):
    def body(d_ref, h1_ref, h2_ref, m_ref, w_ref, b_ref, o_ref):
        rep = jnp.dot(d_ref[...], w_ref[0:D, :],
                      preferred_element_type=jnp.float32)
        rep += jnp.dot(h1_ref[0], w_ref[D:2 * D, :],
                       preferred_element_type=jnp.float32)
        rep += jnp.dot(h2_ref[0], w_ref[2 * D:3 * D, :],
                       preferred_element_type=jnp.float32)
        rep += jnp.dot(h1_ref[1], w_ref[3 * D:4 * D, :],
                       preferred_element_type=jnp.float32)
        rep += jnp.dot(h2_ref[1], w_ref[4 * D:5 * D, :],
                       preferred_element_type=jnp.float32)
        rep = jnp.where(m_ref[...] > 0.0, rep, 0.0) + b_ref[...]
        mx = jnp.max(rep, axis=-1, keepdims=True)
        lse = jnp.log(jnp.sum(jnp.exp(rep - mx), axis=-1, keepdims=True)) + mx
        o_ref[...] = rep - lse

    return pl.pallas_call(
        body,
        grid=(NUM_ROW_BLOCKS,),
        in_specs=[
            pl.BlockSpec((ROW_BLOCK, D), lambda i: (i, 0)),
            pl.BlockSpec((2, ROW_BLOCK, D), lambda i: (0, i, 0)),
            pl.BlockSpec((2, ROW_BLOCK, D), lambda i: (0, i, 0)),
            pl.BlockSpec((ROW_BLOCK, 1), lambda i: (i, 0)),
            pl.BlockSpec((5 * D, NUM_CLASSES), lambda i: (0, 0)),
            pl.BlockSpec((1, NUM_CLASSES), lambda i: (0, 0)),
        ],
        out_specs=pl.BlockSpec((ROW_BLOCK, NUM_CLASSES), lambda i: (i, 0)),
        out_shape=jax.ShapeDtypeStruct((N, NUM_CLASSES), jnp.float32),
    )(dense, h1, h2, maskf, fc2_w, fc2_b2d)


def kernel(x, edge_index, mask, fc1_w, fc1_b, mw_0_0, mw_0_1, mw_1_0, mw_1_1,
           rw_0_0, rw_0_1, rw_1_0, rw_1_1, fc2_w, fc2_b):
    src = edge_index[0]
    dst = edge_index[1]
    pad = E_PAD - E
    zpad = jnp.zeros((pad,), jnp.int32)
    npad = jnp.full((pad,), N, jnp.int32)
    src_p = jnp.concatenate([src, zpad])
    dst_p = jnp.concatenate([dst, zpad])
    idx_shape = (2, NUM_TILES, CHUNKS_PER_TILE, CHUNK)
    gidx2 = jnp.stack([src_p, dst_p + N]).reshape(idx_shape)
    sidx = jnp.stack(
        [jnp.concatenate([dst, npad]), jnp.concatenate([src, npad])]
    ).reshape(idx_shape)
    z128 = jnp.zeros((CHUNK, D), jnp.float32)
    ones128 = jnp.ones((CHUNK, D), jnp.float32)
    maskf = mask.astype(jnp.float32).reshape(N, 1)

    dense2 = _dense_layer(x, fc1_w, fc1_b.reshape(1, D))
    deg = _degrees(sidx, z128, ones128)
    agg1 = _propagate(dense2.reshape(2 * N, D), gidx2, sidx, z128)
    h1 = _mlp_pair(dense2, agg1, deg,
                   jnp.stack([mw_0_0, rw_0_0]), jnp.stack([mw_0_1, rw_0_1]))
    agg2 = _propagate(h1.reshape(2 * N, D), gidx2, sidx, z128)
    h2 = _mlp_pair(h1, agg2, deg,
                   jnp.stack([mw_1_0, rw_1_0]), jnp.stack([mw_1_1, rw_1_1]))
    return _final_layer(dense2[0], h1, h2, maskf, fc2_w,
                        fc2_b.reshape(1, NUM_CLASSES))


# speedup vs baseline: 1.8074x; 1.0107x over previous
import functools

import jax
import jax.numpy as jnp
from jax import lax
from jax.experimental import pallas as pl
from jax.experimental.pallas import tpu as pltpu
from jax.experimental.pallas import tpu_sc as plsc

N = 10000
E = 320000
D = 128
NUM_CLASSES = 16

NUM_TILES = 16
CHUNK = 120
IDX_GRP = 8
CHUNKS_PER_TILE = 168
IDX_GRPS = CHUNKS_PER_TILE // IDX_GRP
TILE_EDGES = CHUNKS_PER_TILE * CHUNK
E_PAD = NUM_TILES * TILE_EDGES
N_PAD = 10112
ROWS_PER_TILE = N_PAD // NUM_TILES
_ROW_SPANS = [(q, min(CHUNK, ROWS_PER_TILE - q))
              for q in range(0, ROWS_PER_TILE, CHUNK)]
ROW_BLOCK = 1000
NUM_ROW_BLOCKS = N // ROW_BLOCK


def _sc_mesh():
    return plsc.VectorSubcoreMesh(core_axis_name="c", subcore_axis_name="s")


def _degrees(sidx, z128, ones128):
    @functools.partial(
        pl.kernel,
        out_type=jax.ShapeDtypeStruct((2, N_PAD, D), jnp.float32),
        mesh=_sc_mesh(),
        scratch_types=[
            pltpu.VMEM((IDX_GRP, CHUNK), jnp.int32),
            pltpu.VMEM((CHUNK, D), jnp.float32),
            pltpu.VMEM((CHUNK, D), jnp.float32),
            pltpu.VMEM_SHARED((N_PAD, D), jnp.float32),
            pltpu.SemaphoreType.DMA,
        ],
    )
    def body(sidx_h, z128_h, ones_h, deg_h, idxs_v, rows_v, ones_v, deg_sh,
             ssem):
        c = lax.axis_index("c")
        s = lax.axis_index("s")
        row0 = s * ROWS_PER_TILE
        pltpu.sync_copy(z128_h, rows_v)
        pltpu.sync_copy(ones_h, ones_v)
        for q0, qn in _ROW_SPANS:
            pltpu.sync_copy(rows_v.at[pl.ds(0, qn)],
                            deg_sh.at[pl.ds(row0 + q0, qn)])
        plsc.subcore_barrier()

        @pl.loop(0, IDX_GRPS)
        def _(g):
            pltpu.sync_copy(sidx_h.at[c, s, pl.ds(g * IDX_GRP, IDX_GRP)], idxs_v)
            descs = []
            for k in range(IDX_GRP):
                descs.append(pltpu.async_copy(
                    ones_v, deg_sh.at[idxs_v.at[k]], ssem, add=True))
            for d in descs:
                d.wait()

        plsc.subcore_barrier()
        for q0, qn in _ROW_SPANS:
            pltpu.sync_copy(deg_sh.at[pl.ds(row0 + q0, qn)], rows_v.at[pl.ds(0, qn)])
            pltpu.sync_copy(rows_v.at[pl.ds(0, qn)],
                            deg_h.at[c, pl.ds(row0 + q0, qn)])

    return body(sidx, z128, ones128)


def _propagate(table2, gidx, sidx, z128):
    @functools.partial(
        pl.kernel,
        out_type=jax.ShapeDtypeStruct((2, N_PAD, D), jnp.float32),
        mesh=_sc_mesh(),
        scratch_types=[
            pltpu.VMEM((IDX_GRP, CHUNK), jnp.int32),
            pltpu.VMEM((IDX_GRP, CHUNK), jnp.int32),
            pltpu.VMEM((3, CHUNK, D), jnp.float32),
            pltpu.VMEM_SHARED((N_PAD, D), jnp.float32),
            pltpu.SemaphoreType.DMA,
            pltpu.SemaphoreType.DMA,
            pltpu.SemaphoreType.DMA,
            pltpu.SemaphoreType.DMA,
            pltpu.SemaphoreType.DMA,
            pltpu.SemaphoreType.DMA,
        ],
    )
    def body(table_h, gidx_h, sidx_h, z128_h, agg_h,
             idxg_v, idxs_v, rows_v, acc_sh,
             gsem0, gsem1, gsem2, ssem0, ssem1, ssem2):
        c = lax.axis_index("c")
        s = lax.axis_index("s")
        row0 = s * ROWS_PER_TILE
        gsems = (gsem0, gsem1, gsem2)
        ssems = (ssem0, ssem1, ssem2)
        pltpu.sync_copy(z128_h, rows_v.at[0])
        for q0, qn in _ROW_SPANS:
            pltpu.sync_copy(rows_v.at[0, pl.ds(0, qn)],
                            acc_sh.at[pl.ds(row0 + q0, qn)])
        plsc.subcore_barrier()

        @pl.loop(0, IDX_GRPS)
        def _(g):
            pltpu.sync_copy(gidx_h.at[c, s, pl.ds(g * IDX_GRP, IDX_GRP)], idxg_v)
            pltpu.sync_copy(sidx_h.at[c, s, pl.ds(g * IDX_GRP, IDX_GRP)], idxs_v)
            def fire_gather(k, b):
                return pltpu.async_copy(
                    table_h.at[idxg_v.at[k]], rows_v.at[b], gsems[b])

            gath = [None] * IDX_GRP
            scat = [None, None, None]
            gath[0] = fire_gather(0, 0)
            gath[1] = fire_gather(1, 1)
            for k in range(IDX_GRP):
                b = k % 3
                gath[k].wait()
                if k + 2 < IDX_GRP:
                    b2 = (k + 2) % 3
                    if scat[b2] is not None:
                        scat[b2].wait()
                        scat[b2] = None
                    gath[k + 2] = fire_gather(k + 2, b2)
                scat[b] = pltpu.async_copy(
                    rows_v.at[b], acc_sh.at[idxs_v.at[k]], ssems[b], add=True)
            for d in scat:
                if d is not None:
                    d.wait()

        plsc.subcore_barrier()
        for q0, qn in _ROW_SPANS:
            pltpu.sync_copy(acc_sh.at[pl.ds(row0 + q0, qn)],
                            rows_v.at[0, pl.ds(0, qn)])
            pltpu.sync_copy(rows_v.at[0, pl.ds(0, qn)],
                            agg_h.at[c, pl.ds(row0 + q0, qn)])

    return body(table2, gidx, sidx, z128)


def _dense_layer(x, w, b2d):
    def body(x_ref, w_ref, b_ref, o_ref):
        acc = jnp.dot(x_ref[...], w_ref[...],
                      preferred_element_type=jnp.float32)
        v = jnp.maximum(acc + b_ref[...], 0.0)
        o_ref[0] = v
        o_ref[1] = v

    return pl.pallas_call(
        body,
        grid=(NUM_ROW_BLOCKS,),
        in_specs=[
            pl.BlockSpec((ROW_BLOCK, D), lambda i: (i, 0)),
            pl.BlockSpec((D, D), lambda i: (0, 0)),
            pl.BlockSpec((1, D), lambda i: (0, 0)),
        ],
        out_specs=pl.BlockSpec((2, ROW_BLOCK, D), lambda i: (0, i, 0)),
        out_shape=jax.ShapeDtypeStruct((2, N, D), jnp.float32),
    )(x, w, b2d)


def _mlp_pair(h_prev, agg, deg, w0s, w1s):
    hd = h_prev.shape[0]

    def body(h_ref, a_ref, g_ref, w0_ref, w1_ref, o_ref):
        h = h_ref[0]
        a = a_ref[0]
        inv = 1.0 / jnp.maximum(g_ref[0][:, 0:1], 1.0)
        an = a * inv
        t = jnp.dot(h, w0_ref[0, :D, :], preferred_element_type=jnp.float32)
        t += jnp.dot(an, w0_ref[0, D:, :], preferred_element_type=jnp.float32)
        t = jnp.maximum(t, 0.0)
        o_ref[0] = jnp.maximum(
            jnp.dot(t, w1_ref[0], preferred_element_type=jnp.float32), 0.0)

    return pl.pallas_call(
        body,
        grid=(2, NUM_ROW_BLOCKS),
        in_specs=[
            pl.BlockSpec((1, ROW_BLOCK, D),
                         lambda d, i: (d if hd == 2 else 0, i, 0)),
            pl.BlockSpec((1, ROW_BLOCK, D), lambda d, i: (d, i, 0)),
            pl.BlockSpec((1, ROW_BLOCK, D), lambda d, i: (d, i, 0)),
            pl.BlockSpec((1, 2 * D, D), lambda d, i: (d, 0, 0)),
            pl.BlockSpec((1, D, D), lambda d, i: (d, 0, 0)),
        ],
        out_specs=pl.BlockSpec((1, ROW_BLOCK, D), lambda d, i: (d, i, 0)),
        out_shape=jax.ShapeDtypeStruct((2, N, D), jnp.float32),
    )(h_prev, agg, deg, w0s, w1s)


def _mlp2_final(dense, h1, agg2, deg, w0s, w1s, maskf, fc2_w, fc2_b2d):
    def body(d_ref, h1_ref, a_ref, g_ref, w0_ref, w1_ref, m_ref, w2_ref,
             b_ref, o_ref):
        h2 = []
        for d in range(2):
            inv = 1.0 / jnp.maximum(g_ref[d][:, 0:1], 1.0)
            an = a_ref[d] * inv
            t = jnp.dot(h1_ref[d], w0_ref[d, :D, :],
                        preferred_element_type=jnp.float32)
            t += jnp.dot(an, w0_ref[d, D:, :],
                         preferred_element_type=jnp.float32)
            t = jnp.maximum(t, 0.0)
            h2.append(jnp.maximum(
                jnp.dot(t, w1_ref[d], preferred_element_type=jnp.float32),
                0.0))
        rep = jnp.dot(d_ref[...], w2_ref[0:D, :],
                      preferred_element_type=jnp.float32)
        rep += jnp.dot(h1_ref[0], w2_ref[D:2 * D, :],
                       preferred_element_type=jnp.float32)
        rep += jnp.dot(h2[0], w2_ref[2 * D:3 * D, :],
                       preferred_element_type=jnp.float32)
        rep += jnp.dot(h1_ref[1], w2_ref[3 * D:4 * D, :],
                       preferred_element_type=jnp.float32)
        rep += jnp.dot(h2[1], w2_ref[4 * D:5 * D, :],
                       preferred_element_type=jnp.float32)
        rep = jnp.where(m_ref[...] > 0.0, rep, 0.0) + b_ref[...]
        mx = jnp.max(rep, axis=-1, keepdims=True)
        lse = jnp.log(jnp.sum(jnp.exp(rep - mx), axis=-1, keepdims=True)) + mx
        o_ref[...] = rep - lse

    return pl.pallas_call(
        body,
        grid=(NUM_ROW_BLOCKS,),
        in_specs=[
            pl.BlockSpec((ROW_BLOCK, D), lambda i: (i, 0)),
            pl.BlockSpec((2, ROW_BLOCK, D), lambda i: (0, i, 0)),
            pl.BlockSpec((2, ROW_BLOCK, D), lambda i: (0, i, 0)),
            pl.BlockSpec((2, ROW_BLOCK, D), lambda i: (0, i, 0)),
            pl.BlockSpec((2, 2 * D, D), lambda i: (0, 0, 0)),
            pl.BlockSpec((2, D, D), lambda i: (0, 0, 0)),
            pl.BlockSpec((ROW_BLOCK, 1), lambda i: (i, 0)),
            pl.BlockSpec((5 * D, NUM_CLASSES), lambda i: (0, 0)),
            pl.BlockSpec((1, NUM_CLASSES), lambda i: (0, 0)),
        ],
        out_specs=pl.BlockSpec((ROW_BLOCK, NUM_CLASSES), lambda i: (i, 0)),
        out_shape=jax.ShapeDtypeStruct((N, NUM_CLASSES), jnp.float32),
    )(dense, h1, agg2, deg, w0s, w1s, maskf, fc2_w, fc2_b2d)


def kernel(x, edge_index, mask, fc1_w, fc1_b, mw_0_0, mw_0_1, mw_1_0, mw_1_1,
           rw_0_0, rw_0_1, rw_1_0, rw_1_1, fc2_w, fc2_b):
    src = edge_index[0]
    dst = edge_index[1]
    pad = E_PAD - E
    zpad = jnp.zeros((pad,), jnp.int32)
    npad = jnp.full((pad,), N, jnp.int32)
    src_p = jnp.concatenate([src, zpad])
    dst_p = jnp.concatenate([dst, zpad])
    idx_shape = (2, NUM_TILES, CHUNKS_PER_TILE, CHUNK)
    gidx2 = jnp.stack([src_p, dst_p + N]).reshape(idx_shape)
    sidx = jnp.stack(
        [jnp.concatenate([dst, npad]), jnp.concatenate([src, npad])]
    ).reshape(idx_shape)
    z128 = jnp.zeros((CHUNK, D), jnp.float32)
    ones128 = jnp.ones((CHUNK, D), jnp.float32)
    maskf = mask.astype(jnp.float32).reshape(N, 1)

    dense2 = _dense_layer(x, fc1_w, fc1_b.reshape(1, D))
    deg = _degrees(sidx, z128, ones128)
    agg1 = _propagate(dense2.reshape(2 * N, D), gidx2, sidx, z128)
    h1 = _mlp_pair(dense2, agg1, deg,
                   jnp.stack([mw_0_0, rw_0_0]), jnp.stack([mw_0_1, rw_0_1]))
    agg2 = _propagate(h1.reshape(2 * N, D), gidx2, sidx, z128)
    return _mlp2_final(dense2[0], h1, agg2, deg,
                       jnp.stack([mw_1_0, rw_1_0]), jnp.stack([mw_1_1, rw_1_1]),
                       maskf, fc2_w, fc2_b.reshape(1, NUM_CLASSES))


# speedup vs baseline: 1.8168x; 1.0052x over previous
import functools

import jax
import jax.numpy as jnp
from jax import lax
from jax.experimental import pallas as pl
from jax.experimental.pallas import tpu as pltpu
from jax.experimental.pallas import tpu_sc as plsc

N = 10000
E = 320000
D = 128
NUM_CLASSES = 16

NUM_TILES = 16
CHUNK = 120
IDX_GRP = 8
CHUNKS_PER_TILE = 168
IDX_GRPS = CHUNKS_PER_TILE // IDX_GRP
TILE_EDGES = CHUNKS_PER_TILE * CHUNK
E_PAD = NUM_TILES * TILE_EDGES
N_PAD = 10112
ROWS_PER_TILE = N_PAD // NUM_TILES
_ROW_SPANS = [(q, min(CHUNK, ROWS_PER_TILE - q))
              for q in range(0, ROWS_PER_TILE, CHUNK)]
ROW_BLOCK = 1000
NUM_ROW_BLOCKS = N // ROW_BLOCK


def _sc_mesh():
    return plsc.VectorSubcoreMesh(core_axis_name="c", subcore_axis_name="s")


def _propagate(table2, gidx, sidx, z128, ones128=None):
    with_deg = ones128 is not None
    agg_t = jax.ShapeDtypeStruct((2, N_PAD, D), jnp.float32)
    out_type = [agg_t, agg_t] if with_deg else agg_t

    @functools.partial(
        pl.kernel,
        out_type=out_type,
        mesh=_sc_mesh(),
        scratch_types=[
            pltpu.VMEM((IDX_GRP, CHUNK), jnp.int32),
            pltpu.VMEM((IDX_GRP, CHUNK), jnp.int32),
            pltpu.VMEM((3, CHUNK, D), jnp.float32),
            pltpu.VMEM_SHARED((N_PAD, D), jnp.float32),
            pltpu.SemaphoreType.DMA,
            pltpu.SemaphoreType.DMA,
            pltpu.SemaphoreType.DMA,
            pltpu.SemaphoreType.DMA,
            pltpu.SemaphoreType.DMA,
            pltpu.SemaphoreType.DMA,
        ],
    )
    def body(*refs):
        if with_deg:
            (table_h, gidx_h, sidx_h, z128_h, ones_h, agg_h, deg_h,
             idxg_v, idxs_v, rows_v, acc_sh,
             gsem0, gsem1, gsem2, ssem0, ssem1, ssem2) = refs
        else:
            (table_h, gidx_h, sidx_h, z128_h, agg_h,
             idxg_v, idxs_v, rows_v, acc_sh,
             gsem0, gsem1, gsem2, ssem0, ssem1, ssem2) = refs
        c = lax.axis_index("c")
        s = lax.axis_index("s")
        row0 = s * ROWS_PER_TILE
        gsems = (gsem0, gsem1, gsem2)
        ssems = (ssem0, ssem1, ssem2)
        pltpu.sync_copy(z128_h, rows_v.at[0])
        for q0, qn in _ROW_SPANS:
            pltpu.sync_copy(rows_v.at[0, pl.ds(0, qn)],
                            acc_sh.at[pl.ds(row0 + q0, qn)])
        plsc.subcore_barrier()

        @pl.loop(0, IDX_GRPS)
        def _(g):
            pltpu.sync_copy(gidx_h.at[c, s, pl.ds(g * IDX_GRP, IDX_GRP)], idxg_v)
            pltpu.sync_copy(sidx_h.at[c, s, pl.ds(g * IDX_GRP, IDX_GRP)], idxs_v)
            def fire_gather(k, b):
                return pltpu.async_copy(
                    table_h.at[idxg_v.at[k]], rows_v.at[b], gsems[b])

            gath = [None] * IDX_GRP
            scat = [None, None, None]
            gath[0] = fire_gather(0, 0)
            gath[1] = fire_gather(1, 1)
            for k in range(IDX_GRP):
                b = k % 3
                gath[k].wait()
                if k + 2 < IDX_GRP:
                    b2 = (k + 2) % 3
                    if scat[b2] is not None:
                        scat[b2].wait()
                        scat[b2] = None
                    gath[k + 2] = fire_gather(k + 2, b2)
                scat[b] = pltpu.async_copy(
                    rows_v.at[b], acc_sh.at[idxs_v.at[k]], ssems[b], add=True)
            for d in scat:
                if d is not None:
                    d.wait()

        plsc.subcore_barrier()
        for q0, qn in _ROW_SPANS:
            pltpu.sync_copy(acc_sh.at[pl.ds(row0 + q0, qn)],
                            rows_v.at[0, pl.ds(0, qn)])
            pltpu.sync_copy(rows_v.at[0, pl.ds(0, qn)],
                            agg_h.at[c, pl.ds(row0 + q0, qn)])

        if with_deg:
            plsc.subcore_barrier()
            pltpu.sync_copy(z128_h, rows_v.at[0])
            pltpu.sync_copy(ones_h, rows_v.at[1])
            for q0, qn in _ROW_SPANS:
                pltpu.sync_copy(rows_v.at[0, pl.ds(0, qn)],
                                acc_sh.at[pl.ds(row0 + q0, qn)])
            plsc.subcore_barrier()

            @pl.loop(0, IDX_GRPS)
            def _(g):
                pltpu.sync_copy(sidx_h.at[c, s, pl.ds(g * IDX_GRP, IDX_GRP)],
                                idxs_v)
                descs = []
                for k in range(IDX_GRP):
                    descs.append(pltpu.async_copy(
                        rows_v.at[1], acc_sh.at[idxs_v.at[k]], ssems[0],
                        add=True))
                for d in descs:
                    d.wait()

            plsc.subcore_barrier()
            for q0, qn in _ROW_SPANS:
                pltpu.sync_copy(acc_sh.at[pl.ds(row0 + q0, qn)],
                                rows_v.at[0, pl.ds(0, qn)])
                pltpu.sync_copy(rows_v.at[0, pl.ds(0, qn)],
                                deg_h.at[c, pl.ds(row0 + q0, qn)])

    if with_deg:
        return body(table2, gidx, sidx, z128, ones128)
    return body(table2, gidx, sidx, z128)


def _dense_layer(x, w, b2d):
    def body(x_ref, w_ref, b_ref, o_ref):
        acc = jnp.dot(x_ref[...], w_ref[...],
                      preferred_element_type=jnp.float32)
        v = jnp.maximum(acc + b_ref[...], 0.0)
        o_ref[0] = v
        o_ref[1] = v

    return pl.pallas_call(
        body,
        grid=(NUM_ROW_BLOCKS,),
        in_specs=[
            pl.BlockSpec((ROW_BLOCK, D), lambda i: (i, 0)),
            pl.BlockSpec((D, D), lambda i: (0, 0)),
            pl.BlockSpec((1, D), lambda i: (0, 0)),
        ],
        out_specs=pl.BlockSpec((2, ROW_BLOCK, D), lambda i: (0, i, 0)),
        out_shape=jax.ShapeDtypeStruct((2, N, D), jnp.float32),
    )(x, w, b2d)


def _mlp_pair(h_prev, agg, deg, w0s, w1s):
    hd = h_prev.shape[0]

    def body(h_ref, a_ref, g_ref, w0_ref, w1_ref, o_ref):
        h = h_ref[0]
        a = a_ref[0]
        inv = 1.0 / jnp.maximum(g_ref[0][:, 0:1], 1.0)
        an = a * inv
        t = jnp.dot(h, w0_ref[0, :D, :], preferred_element_type=jnp.float32)
        t += jnp.dot(an, w0_ref[0, D:, :], preferred_element_type=jnp.float32)
        t = jnp.maximum(t, 0.0)
        o_ref[0] = jnp.maximum(
            jnp.dot(t, w1_ref[0], preferred_element_type=jnp.float32), 0.0)

    return pl.pallas_call(
        body,
        grid=(2, NUM_ROW_BLOCKS),
        in_specs=[
            pl.BlockSpec((1, ROW_BLOCK, D),
                         lambda d, i: (d if hd == 2 else 0, i, 0)),
            pl.BlockSpec((1, ROW_BLOCK, D), lambda d, i: (d, i, 0)),
            pl.BlockSpec((1, ROW_BLOCK, D), lambda d, i: (d, i, 0)),
            pl.BlockSpec((1, 2 * D, D), lambda d, i: (d, 0, 0)),
            pl.BlockSpec((1, D, D), lambda d, i: (d, 0, 0)),
        ],
        out_specs=pl.BlockSpec((1, ROW_BLOCK, D), lambda d, i: (d, i, 0)),
        out_shape=jax.ShapeDtypeStruct((2, N, D), jnp.float32),
    )(h_prev, agg, deg, w0s, w1s)


def _mlp2_final(dense, h1, agg2, deg, w0s, w1s, maskf, fc2_w, fc2_b2d):
    def body(d_ref, h1_ref, a_ref, g_ref, w0_ref, w1_ref, m_ref, w2_ref,
             b_ref, o_ref):
        h2 = []
        for d in range(2):
            inv = 1.0 / jnp.maximum(g_ref[d][:, 0:1], 1.0)
            an = a_ref[d] * inv
            t = jnp.dot(h1_ref[d], w0_ref[d, :D, :],
                        preferred_element_type=jnp.float32)
            t += jnp.dot(an, w0_ref[d, D:, :],
                         preferred_element_type=jnp.float32)
            t = jnp.maximum(t, 0.0)
            h2.append(jnp.maximum(
                jnp.dot(t, w1_ref[d], preferred_element_type=jnp.float32),
                0.0))
        rep = jnp.dot(d_ref[...], w2_ref[0:D, :],
                      preferred_element_type=jnp.float32)
        rep += jnp.dot(h1_ref[0], w2_ref[D:2 * D, :],
                       preferred_element_type=jnp.float32)
        rep += jnp.dot(h2[0], w2_ref[2 * D:3 * D, :],
                       preferred_element_type=jnp.float32)
        rep += jnp.dot(h1_ref[1], w2_ref[3 * D:4 * D, :],
                       preferred_element_type=jnp.float32)
        rep += jnp.dot(h2[1], w2_ref[4 * D:5 * D, :],
                       preferred_element_type=jnp.float32)
        rep = jnp.where(m_ref[...] > 0.0, rep, 0.0) + b_ref[...]
        mx = jnp.max(rep, axis=-1, keepdims=True)
        lse = jnp.log(jnp.sum(jnp.exp(rep - mx), axis=-1, keepdims=True)) + mx
        o_ref[...] = rep - lse

    return pl.pallas_call(
        body,
        grid=(NUM_ROW_BLOCKS,),
        in_specs=[
            pl.BlockSpec((ROW_BLOCK, D), lambda i: (i, 0)),
            pl.BlockSpec((2, ROW_BLOCK, D), lambda i: (0, i, 0)),
            pl.BlockSpec((2, ROW_BLOCK, D), lambda i: (0, i, 0)),
            pl.BlockSpec((2, ROW_BLOCK, D), lambda i: (0, i, 0)),
            pl.BlockSpec((2, 2 * D, D), lambda i: (0, 0, 0)),
            pl.BlockSpec((2, D, D), lambda i: (0, 0, 0)),
            pl.BlockSpec((ROW_BLOCK, 1), lambda i: (i, 0)),
            pl.BlockSpec((5 * D, NUM_CLASSES), lambda i: (0, 0)),
            pl.BlockSpec((1, NUM_CLASSES), lambda i: (0, 0)),
        ],
        out_specs=pl.BlockSpec((ROW_BLOCK, NUM_CLASSES), lambda i: (i, 0)),
        out_shape=jax.ShapeDtypeStruct((N, NUM_CLASSES), jnp.float32),
    )(dense, h1, agg2, deg, w0s, w1s, maskf, fc2_w, fc2_b2d)


def kernel(x, edge_index, mask, fc1_w, fc1_b, mw_0_0, mw_0_1, mw_1_0, mw_1_1,
           rw_0_0, rw_0_1, rw_1_0, rw_1_1, fc2_w, fc2_b):
    src = edge_index[0]
    dst = edge_index[1]
    pad = E_PAD - E
    zpad = jnp.zeros((pad,), jnp.int32)
    npad = jnp.full((pad,), N, jnp.int32)
    src_p = jnp.concatenate([src, zpad])
    dst_p = jnp.concatenate([dst, zpad])
    idx_shape = (2, NUM_TILES, CHUNKS_PER_TILE, CHUNK)
    gidx2 = jnp.stack([src_p, dst_p + N]).reshape(idx_shape)
    sidx = jnp.stack(
        [jnp.concatenate([dst, npad]), jnp.concatenate([src, npad])]
    ).reshape(idx_shape)
    z128 = jnp.zeros((CHUNK, D), jnp.float32)
    ones128 = jnp.ones((CHUNK, D), jnp.float32)
    maskf = mask.astype(jnp.float32).reshape(N, 1)

    dense2 = _dense_layer(x, fc1_w, fc1_b.reshape(1, D))
    agg1, deg = _propagate(dense2.reshape(2 * N, D), gidx2, sidx, z128, ones128)
    h1 = _mlp_pair(dense2, agg1, deg,
                   jnp.stack([mw_0_0, rw_0_0]), jnp.stack([mw_0_1, rw_0_1]))
    agg2 = _propagate(h1.reshape(2 * N, D), gidx2, sidx, z128)
    return _mlp2_final(dense2[0], h1, agg2, deg,
                       jnp.stack([mw_1_0, rw_1_0]), jnp.stack([mw_1_1, rw_1_1]),
                       maskf, fc2_w, fc2_b.reshape(1, NUM_CLASSES))
